# baseline (device time: 106462 ns/iter reference)
import jax
import jax.numpy as jnp
from jax import lax
from jax.experimental import pallas as pl
from jax.experimental.pallas import tpu as pltpu

B, SQ, H, D = 4, 256, 16, 64
HD = H * D
ROWS = B * SQ
SCALE = D ** -0.5

C = 8
CH = ROWS // C
CHS = SQ // (C // B)


def _comm_body(k_ref, v_ref, kr_ref, vr_ref, sendbuf, krbuf, vrbuf,
               s1, r1, s2, r2, ybar, xbar):
    my_x = lax.axis_index("x")
    my_y = lax.axis_index("y")
    ynbr = (my_x, 1 - my_y)
    xnbr = (1 - my_x, my_y)

    pl.semaphore_signal(ybar, inc=1, device_id=ynbr,
                        device_id_type=pl.DeviceIdType.MESH)
    pl.semaphore_signal(xbar, inc=1, device_id=xnbr,
                        device_id_type=pl.DeviceIdType.MESH)
    pl.semaphore_wait(ybar, 1)
    pl.semaphore_wait(xbar, 1)

    def run(src_ref, recv1, recv2, out1, out2):
        mk = lambda **kw: pltpu.make_async_remote_copy(
            device_id_type=pl.DeviceIdType.MESH, **kw)
        ch = lambda ref, i: ref.at[pl.ds(i * CH, CH)]
        p1 = [mk(src_ref=ch(sendbuf, i), dst_ref=ch(recv1, i),
                 send_sem=s1.at[i], recv_sem=r1.at[i], device_id=ynbr)
              for i in range(C)]
        fwd = [mk(src_ref=ch(recv1, i), dst_ref=ch(recv1, i),
                  send_sem=s2.at[i], recv_sem=r2.at[i], device_id=xnbr)
               for i in range(C)]
        p2w = [mk(src_ref=ch(sendbuf, i), dst_ref=ch(recv2, i),
                  send_sem=s1.at[i], recv_sem=r2.at[i], device_id=xnbr)
               for i in range(C)]
        for i in range(C):
            b, s = divmod(i, C // B)
            sendbuf[pl.ds(i * CH, CH), :] = (
                src_ref[b, pl.ds(s * CHS, CHS), :, :]
                .astype(jnp.bfloat16)
                .reshape(CH, HD)
            )
            p1[i].start()
        for i in range(C):
            p1[i].wait_recv()
            fwd[i].start()
            out1[pl.ds(i * CH, CH), :] = recv1[pl.ds(i * CH, CH), :]
        for i in range(C):
            p2w[i].wait_recv()
            out2[pl.ds(i * CH, CH), :] = recv2[pl.ds(i * CH, CH), :]
        for i in range(C):
            p1[i].wait_send()
            fwd[i].wait_send()

    @pl.when(my_x == 0)
    def _():
        run(k_ref, krbuf, vrbuf, kr_ref, vr_ref)

    @pl.when(my_x == 1)
    def _():
        run(v_ref, vrbuf, krbuf, vr_ref, kr_ref)


def _exchange(K, V):
    return pl.pallas_call(
        _comm_body,
        out_shape=(
            jax.ShapeDtypeStruct((ROWS, HD), jnp.bfloat16),
            jax.ShapeDtypeStruct((ROWS, HD), jnp.bfloat16),
        ),
        in_specs=[pl.BlockSpec(memory_space=pltpu.VMEM)] * 2,
        out_specs=(pl.BlockSpec(memory_space=pltpu.VMEM),) * 2,
        scratch_shapes=[
            pltpu.VMEM((ROWS, HD), jnp.bfloat16),
            pltpu.VMEM((ROWS, HD), jnp.bfloat16),
            pltpu.VMEM((ROWS, HD), jnp.bfloat16),
            pltpu.SemaphoreType.DMA((C,)),
            pltpu.SemaphoreType.DMA((C,)),
            pltpu.SemaphoreType.DMA((C,)),
            pltpu.SemaphoreType.DMA((C,)),
            pltpu.SemaphoreType.REGULAR,
            pltpu.SemaphoreType.REGULAR,
        ],
    )(K, V)


def _one_head(q, kl, vl, kr, vr):
    nt = (((1,), (1,)), ((), ()))
    nn = (((1,), (0,)), ((), ()))
    s1 = lax.dot_general(q, kl, nt, preferred_element_type=jnp.float32) * SCALE
    s2 = lax.dot_general(q, kr, nt, preferred_element_type=jnp.float32) * SCALE
    m = jnp.maximum(jnp.max(s1, axis=1, keepdims=True),
                    jnp.max(s2, axis=1, keepdims=True))
    p1 = jnp.exp(s1 - m)
    p2 = jnp.exp(s2 - m)
    denom = jnp.sum(p1, axis=1, keepdims=True) + jnp.sum(p2, axis=1, keepdims=True)
    o1 = lax.dot_general(p1.astype(jnp.bfloat16), vl, nn,
                         preferred_element_type=jnp.float32)
    o2 = lax.dot_general(p2.astype(jnp.bfloat16), vr, nn,
                         preferred_element_type=jnp.float32)
    return (o1 + o2) / denom


def _attn_body(q_ref, kl_ref, vl_ref, kr_ref, vr_ref, o_ref):
    for h in range(H):
        hs = slice(h * D, (h + 1) * D)
        o_ref[0, :, h, :] = _one_head(
            q_ref[0, :, h, :].astype(jnp.bfloat16),
            kl_ref[0, :, h, :].astype(jnp.bfloat16),
            vl_ref[0, :, h, :].astype(jnp.bfloat16),
            kr_ref[:, hs],
            vr_ref[:, hs],
        )


def kernel(Q, K, V):
    k_rem, v_rem = _exchange(K, V)

    blk4 = lambda: pl.BlockSpec((1, SQ, H, D), lambda b: (b, 0, 0, 0))
    blk2 = lambda: pl.BlockSpec((SQ, HD), lambda b: (b, 0))
    return pl.pallas_call(
        _attn_body,
        grid=(B,),
        in_specs=[blk4(), blk4(), blk4(), blk2(), blk2()],
        out_specs=blk4(),
        out_shape=jax.ShapeDtypeStruct((B, SQ, H, D), jnp.float32),
        compiler_params=pltpu.CompilerParams(
            dimension_semantics=("arbitrary",)),
    )(Q, K, V, k_rem, v_rem)


# device time: 85368 ns/iter; 1.2471x vs baseline; 1.2471x over previous
import jax
import jax.numpy as jnp
from jax import lax
from jax.experimental import pallas as pl
from jax.experimental.pallas import tpu as pltpu

B, SQ, H, D = 4, 256, 16, 64
HD = H * D
ROWS = B * SQ
SCALE = D ** -0.5

C = 8
CH = ROWS // C
CHS = SQ // (C // B)


def _comm_body(k_ref, v_ref, kr_ref, vr_ref, sendbuf, krbuf, vrbuf,
               s1, r1, s2, r2, ybar, xbar):
    my_x = lax.axis_index("x")
    my_y = lax.axis_index("y")
    ynbr = (my_x, 1 - my_y)
    xnbr = (1 - my_x, my_y)

    pl.semaphore_signal(ybar, inc=1, device_id=ynbr,
                        device_id_type=pl.DeviceIdType.MESH)
    pl.semaphore_signal(xbar, inc=1, device_id=xnbr,
                        device_id_type=pl.DeviceIdType.MESH)
    pl.semaphore_wait(ybar, 1)
    pl.semaphore_wait(xbar, 1)

    def run(src_ref, recv1, recv2, out1, out2):
        mk = lambda **kw: pltpu.make_async_remote_copy(
            device_id_type=pl.DeviceIdType.MESH, **kw)
        ch = lambda ref, i: ref.at[pl.ds(i * CH, CH)]
        p1 = [mk(src_ref=ch(sendbuf, i), dst_ref=ch(recv1, i),
                 send_sem=s1.at[i], recv_sem=r1.at[i], device_id=ynbr)
              for i in range(C)]
        fwd = [mk(src_ref=ch(recv1, i), dst_ref=ch(recv1, i),
                  send_sem=s2.at[i], recv_sem=r2.at[i], device_id=xnbr)
               for i in range(C)]
        p2w = [mk(src_ref=ch(sendbuf, i), dst_ref=ch(recv2, i),
                  send_sem=s1.at[i], recv_sem=r2.at[i], device_id=xnbr)
               for i in range(C)]
        for i in range(C):
            b, s = divmod(i, C // B)
            sendbuf[pl.ds(i * CH, CH), :] = (
                src_ref[b, pl.ds(s * CHS, CHS), :, :]
                .astype(jnp.bfloat16)
                .reshape(CH, HD)
            )
            p1[i].start()
        for i in range(C):
            p1[i].wait_recv()
            fwd[i].start()
            out1[pl.ds(i * CH, CH), :] = recv1[pl.ds(i * CH, CH), :]
        for i in range(C):
            p2w[i].wait_recv()
            out2[pl.ds(i * CH, CH), :] = recv2[pl.ds(i * CH, CH), :]
        for i in range(C):
            p1[i].wait_send()
            fwd[i].wait_send()

    @pl.when(my_x == 0)
    def _():
        run(k_ref, krbuf, vrbuf, kr_ref, vr_ref)

    @pl.when(my_x == 1)
    def _():
        run(v_ref, vrbuf, krbuf, vr_ref, kr_ref)


def _exchange(K, V):
    return pl.pallas_call(
        _comm_body,
        out_shape=(
            jax.ShapeDtypeStruct((ROWS, HD), jnp.bfloat16),
            jax.ShapeDtypeStruct((ROWS, HD), jnp.bfloat16),
        ),
        in_specs=[pl.BlockSpec(memory_space=pltpu.VMEM)] * 2,
        out_specs=(pl.BlockSpec(memory_space=pltpu.VMEM),) * 2,
        scratch_shapes=[
            pltpu.VMEM((ROWS, HD), jnp.bfloat16),
            pltpu.VMEM((ROWS, HD), jnp.bfloat16),
            pltpu.VMEM((ROWS, HD), jnp.bfloat16),
            pltpu.SemaphoreType.DMA((C,)),
            pltpu.SemaphoreType.DMA((C,)),
            pltpu.SemaphoreType.DMA((C,)),
            pltpu.SemaphoreType.DMA((C,)),
            pltpu.SemaphoreType.REGULAR,
            pltpu.SemaphoreType.REGULAR,
        ],
    )(K, V)


def _one_head(q, kl, vl, kr, vr):
    nt = (((1,), (1,)), ((), ()))
    nn = (((1,), (0,)), ((), ()))
    s1 = lax.dot_general(q, kl, nt, preferred_element_type=jnp.float32) * SCALE
    s2 = lax.dot_general(q, kr, nt, preferred_element_type=jnp.float32) * SCALE
    m = jnp.maximum(jnp.max(s1, axis=1, keepdims=True),
                    jnp.max(s2, axis=1, keepdims=True))
    p1 = jnp.exp(s1 - m)
    p2 = jnp.exp(s2 - m)
    denom = jnp.sum(p1, axis=1, keepdims=True) + jnp.sum(p2, axis=1, keepdims=True)
    o1 = lax.dot_general(p1.astype(jnp.bfloat16), vl, nn,
                         preferred_element_type=jnp.float32)
    o2 = lax.dot_general(p2.astype(jnp.bfloat16), vr, nn,
                         preferred_element_type=jnp.float32)
    return (o1 + o2) / denom


def _attn_body(q_ref, kl_ref, vl_ref, kr_ref, vr_ref, o_ref):
    q = q_ref[0].astype(jnp.bfloat16)
    kl = kl_ref[0].astype(jnp.bfloat16)
    vl = vl_ref[0].astype(jnp.bfloat16)
    kr = kr_ref[...]
    vr = vr_ref[...]
    outs = []
    for h in range(H):
        hs = slice(h * D, (h + 1) * D)
        outs.append(_one_head(q[:, hs], kl[:, hs], vl[:, hs],
                              kr[:, hs], vr[:, hs]))
    o_ref[0] = jnp.concatenate(outs, axis=1)


def kernel(Q, K, V):
    k_rem, v_rem = _exchange(K, V)

    q3 = Q.reshape(B, SQ, HD)
    kl3 = K.reshape(B, SQ, HD)
    vl3 = V.reshape(B, SQ, HD)
    blk3 = lambda: pl.BlockSpec((1, SQ, HD), lambda b: (b, 0, 0))
    blk2 = lambda: pl.BlockSpec((SQ, HD), lambda b: (b, 0))
    out = pl.pallas_call(
        _attn_body,
        grid=(B,),
        in_specs=[blk3(), blk3(), blk3(), blk2(), blk2()],
        out_specs=blk3(),
        out_shape=jax.ShapeDtypeStruct((B, SQ, HD), jnp.float32),
        compiler_params=pltpu.CompilerParams(
            dimension_semantics=("arbitrary",)),
    )(q3, kl3, vl3, k_rem, v_rem)
    return out.reshape(B, SQ, H, D)
